# baseline (speedup 1.0000x reference)
import functools

import jax
import jax.numpy as jnp
from jax import lax
from jax.experimental import pallas as pl
from jax.experimental.pallas import tpu as pltpu
from jax.experimental.pallas import tpu_sc as plsc

_N = 10000
_E = 320000
_D = 128

_NSC = 2
_NSUB = 16
_NW = _NSC * _NSUB
_CH = 128
_NCHK = _E // _CH
_CPT = _NCHK // _NW
_LEFT = _NCHK - _CPT * _NW
_RPT = 624

_mesh = plsc.VectorSubcoreMesh(core_axis_name="c", subcore_axis_name="s")


@functools.partial(
    pl.kernel,
    out_type=jax.ShapeDtypeStruct((_NSC, _N, 16), jnp.float32),
    mesh=_mesh,
    scratch_types=[
        pltpu.VMEM((_CH,), jnp.int32),
        pltpu.VMEM((_CH,), jnp.int32),
        pltpu.VMEM((_CH, 16), jnp.float32),
        pltpu.VMEM((48, 16), jnp.float32),
        pltpu.VMEM_SHARED((_N, 16), jnp.float32),
        pltpu.SemaphoreType.DMA,
        pltpu.SemaphoreType.DMA,
    ],
)
def _deg_kernel(dst_hbm, out_hbm, diA, diB, ones_v, zer_v, acc_sh,
                semIA, semIB):
    cid = lax.axis_index("c")
    sid = lax.axis_index("s")
    wid = sid * _NSC + cid

    @pl.loop(0, 48)
    def _(i):
        zer_v[i, :] = jnp.zeros((16,), jnp.float32)

    @pl.loop(0, _CH)
    def _(i):
        ones_v[i, :] = jnp.ones((16,), jnp.float32)

    r0 = sid * _RPT

    @pl.loop(0, 13)
    def _(j):
        pltpu.sync_copy(zer_v, acc_sh.at[pl.ds(r0 + j * 48, 48)])

    @pl.when(sid == _NSUB - 1)
    def _():
        pltpu.sync_copy(zer_v.at[pl.ds(0, 16)], acc_sh.at[pl.ds(_N - 16, 16)])

    plsc.subcore_barrier()

    base = wid * _CPT

    def fetch_idx(c, di, sem):
        return pltpu.make_async_copy(dst_hbm.at[pl.ds(c * _CH, _CH)], di, sem)

    def scatter_ones(di):
        pltpu.sync_copy(ones_v, acc_sh.at[di], add=True)

    fetch_idx(base, diA, semIA).start()
    fetch_idx(base + 1, diB, semIB).start()

    @pl.loop(0, _CPT, step=2)
    def _(t):
        fetch_idx(base + t, diA, semIA).wait()
        scatter_ones(diA)

        @pl.when(t + 2 < _CPT)
        def _():
            fetch_idx(base + t + 2, diA, semIA).start()

        fetch_idx(base + t + 1, diB, semIB).wait()
        scatter_ones(diB)

        @pl.when(t + 3 < _CPT)
        def _():
            fetch_idx(base + t + 3, diB, semIB).start()

    @pl.when(wid < _LEFT)
    def _():
        c = _NW * _CPT + wid
        fetch_idx(c, diA, semIA).start()
        fetch_idx(c, diA, semIA).wait()
        scatter_ones(diA)

    plsc.subcore_barrier()

    @pl.loop(0, 13)
    def _(j):
        rr = r0 + j * 48
        pltpu.sync_copy(acc_sh.at[pl.ds(rr, 48)],
                        out_hbm.at[cid, pl.ds(rr, 48), :])

    @pl.when(sid == _NSUB - 1)
    def _():
        pltpu.sync_copy(acc_sh.at[pl.ds(_N - 16, 16)],
                        out_hbm.at[cid, pl.ds(_N - 16, 16), :])


@functools.partial(
    pl.kernel,
    out_type=jax.ShapeDtypeStruct((_NSC, _N, _D), jnp.float32),
    mesh=_mesh,
    scratch_types=[
        pltpu.VMEM((_CH,), jnp.int32),
        pltpu.VMEM((_CH,), jnp.int32),
        pltpu.VMEM((_CH,), jnp.int32),
        pltpu.VMEM((_CH,), jnp.int32),
        pltpu.VMEM((_CH, _D), jnp.float32),
        pltpu.VMEM((_CH, _D), jnp.float32),
        pltpu.VMEM((48, _D), jnp.float32),
        pltpu.VMEM_SHARED((_N, _D), jnp.float32),
        pltpu.SemaphoreType.DMA,
        pltpu.SemaphoreType.DMA,
        pltpu.SemaphoreType.DMA,
        pltpu.SemaphoreType.DMA,
    ],
)
def _segsum_kernel(hs_hbm, src_hbm, dst_hbm, out_hbm,
                   siA, diA, siB, diB, rA, rB, zer_v, acc_sh,
                   semA, semB, semIA, semIB):
    cid = lax.axis_index("c")
    sid = lax.axis_index("s")
    wid = sid * _NSC + cid

    @pl.loop(0, 48)
    def _(i):
        @pl.loop(0, _D // 16)
        def _(j):
            zer_v[i, pl.ds(j * 16, 16)] = jnp.zeros((16,), jnp.float32)

    r0 = sid * _RPT

    @pl.loop(0, 13)
    def _(j):
        pltpu.sync_copy(zer_v, acc_sh.at[pl.ds(r0 + j * 48, 48)])

    @pl.when(sid == _NSUB - 1)
    def _():
        pltpu.sync_copy(zer_v.at[pl.ds(0, 16)], acc_sh.at[pl.ds(_N - 16, 16)])

    plsc.subcore_barrier()

    base = wid * _CPT

    def fetch_idx(c, si, di, sem):
        return (pltpu.make_async_copy(dst_hbm.at[pl.ds(c * _CH, _CH)], di, sem),
                pltpu.make_async_copy(src_hbm.at[pl.ds(c * _CH, _CH)], si, sem))

    def start_idx(c, si, di, sem):
        a, b = fetch_idx(c, si, di, sem)
        a.start()
        b.start()

    def wait_idx(c, si, di, sem):
        a, b = fetch_idx(c, si, di, sem)
        a.wait()
        b.wait()

    def gather(si, rows, sem):
        return pltpu.make_async_copy(hs_hbm.at[si], rows, sem)

    def scatter(rows, di):
        pltpu.sync_copy(rows, acc_sh.at[di], add=True)

    start_idx(base, siA, diA, semIA)
    wait_idx(base, siA, diA, semIA)
    gather(siA, rA, semA).start()
    start_idx(base + 1, siB, diB, semIB)

    @pl.loop(0, _CPT, step=2)
    def _(t):
        gather(siA, rA, semA).wait()
        wait_idx(base + t + 1, siB, diB, semIB)
        gather(siB, rB, semB).start()
        scatter(rA, diA)

        @pl.when(t + 2 < _CPT)
        def _():
            start_idx(base + t + 2, siA, diA, semIA)

        gather(siB, rB, semB).wait()

        @pl.when(t + 2 < _CPT)
        def _():
            wait_idx(base + t + 2, siA, diA, semIA)
            gather(siA, rA, semA).start()
        scatter(rB, diB)

        @pl.when(t + 3 < _CPT)
        def _():
            start_idx(base + t + 3, siB, diB, semIB)

    @pl.when(wid < _LEFT)
    def _():
        c = _NW * _CPT + wid
        start_idx(c, siA, diA, semIA)
        wait_idx(c, siA, diA, semIA)
        pltpu.async_copy(hs_hbm.at[siA], rA, semA).wait()
        scatter(rA, diA)

    plsc.subcore_barrier()

    @pl.loop(0, 13)
    def _(j):
        rr = r0 + j * 48
        pltpu.sync_copy(acc_sh.at[pl.ds(rr, 48)],
                        out_hbm.at[cid, pl.ds(rr, 48), :])

    @pl.when(sid == _NSUB - 1)
    def _():
        pltpu.sync_copy(acc_sh.at[pl.ds(_N - 16, 16)],
                        out_hbm.at[cid, pl.ds(_N - 16, 16), :])


_BLK = 1000


def _tc1_body(x_ref, lx_ref, w_ref, b_ref, xo_ref, h_ref):
    xo = x_ref[...] * lx_ref[...]
    xo_ref[...] = xo
    h_ref[...] = (jnp.dot(xo, w_ref[...], preferred_element_type=jnp.float32)
                  + b_ref[...])


def _tc2_body(d0_ref, d1_ref, h_ref, hs_ref):
    dinv = lax.rsqrt(d0_ref[...] + d1_ref[...] + 1.0)
    hs_ref[...] = h_ref[...] * dinv


def _tc3_body(d0_ref, d1_ref, s0_ref, s1_ref, hs_ref, w_ref, b_ref, out_ref):
    dinv = lax.rsqrt(d0_ref[...] + d1_ref[...] + 1.0)
    t = (s0_ref[...] + s1_ref[...] + hs_ref[...]) * dinv
    h1 = jnp.maximum(t, 0.0)
    out_ref[...] = (jnp.dot(h1, w_ref[...], preferred_element_type=jnp.float32)
                    + b_ref[...]) * dinv


def _tc4_body(d0_ref, d1_ref, s0_ref, s1_ref, hs_ref,
              w1_ref, b1_ref, w2_ref, b2_ref, out_ref):
    dinv = lax.rsqrt(d0_ref[...] + d1_ref[...] + 1.0)
    embed = (s0_ref[...] + s1_ref[...] + hs_ref[...]) * dinv
    hidden = jnp.maximum(
        jnp.dot(embed, w1_ref[...], preferred_element_type=jnp.float32)
        + b1_ref[...], 0.0)
    out_ref[...] = (jnp.dot(hidden, w2_ref[...],
                            preferred_element_type=jnp.float32) + b2_ref[...])


def _row_spec():
    return pl.BlockSpec((_BLK, _D), lambda i: (i, 0))


def _deg_spec():
    return pl.BlockSpec((_BLK, 1), lambda i: (i, 0))


def _full_spec(shape):
    return pl.BlockSpec(shape, lambda i: tuple(0 for _ in shape))


def kernel(x, edge_index, emb1, emb3, learnable_x, cond_Wi, cond_bi, cond_Wo,
           cond_bo, g_W1, g_b1, g_W2, g_b2, c_W1, c_b1, c_W2, c_b2):
    src = edge_index[0]
    dst = edge_index[1]
    n, d = x.shape
    grid = (n // _BLK,)

    degp = _deg_kernel(dst)
    d0 = degp[0, :, 0].reshape(n, 1)
    d1 = degp[1, :, 0].reshape(n, 1)

    x_out, h1_pre = pl.pallas_call(
        _tc1_body,
        grid=grid,
        in_specs=[_row_spec(), _row_spec(),
                  _full_spec((_D, _D)), _full_spec((1, _D))],
        out_specs=[_row_spec(), _row_spec()],
        out_shape=[jax.ShapeDtypeStruct((n, d), jnp.float32)] * 2,
    )(x, learnable_x, g_W1, g_b1.reshape(1, d))

    hs1 = pl.pallas_call(
        _tc2_body,
        grid=grid,
        in_specs=[_deg_spec(), _deg_spec(), _row_spec()],
        out_specs=_row_spec(),
        out_shape=jax.ShapeDtypeStruct((n, d), jnp.float32),
    )(d0, d1, h1_pre)

    s1 = _segsum_kernel(hs1, src, dst)

    hs2 = pl.pallas_call(
        _tc3_body,
        grid=grid,
        in_specs=[_deg_spec(), _deg_spec(), _row_spec(), _row_spec(),
                  _row_spec(), _full_spec((_D, _D)), _full_spec((1, _D))],
        out_specs=_row_spec(),
        out_shape=jax.ShapeDtypeStruct((n, d), jnp.float32),
    )(d0, d1, s1[0], s1[1], hs1, g_W2, g_b2.reshape(1, d))

    s2 = _segsum_kernel(hs2, src, dst)

    nh = c_W1.shape[1]
    nc = c_W2.shape[1]
    logits = pl.pallas_call(
        _tc4_body,
        grid=grid,
        in_specs=[_deg_spec(), _deg_spec(), _row_spec(), _row_spec(),
                  _row_spec(), _full_spec((_D, nh)), _full_spec((1, nh)),
                  _full_spec((nh, nc)), _full_spec((1, nc))],
        out_specs=pl.BlockSpec((_BLK, nc), lambda i: (i, 0)),
        out_shape=jax.ShapeDtypeStruct((n, nc), jnp.float32),
    )(d0, d1, s2[0], s2[1], hs2, c_W1, c_b1.reshape(1, nh),
      c_W2, c_b2.reshape(1, nc))

    return (x_out, logits)


# speedup vs baseline: 1.0179x; 1.0179x over previous
import functools

import jax
import jax.numpy as jnp
from jax import lax
from jax.experimental import pallas as pl
from jax.experimental.pallas import tpu as pltpu
from jax.experimental.pallas import tpu_sc as plsc

_N = 10000
_E = 320000
_D = 128

_NSC = 2
_NSUB = 16
_NW = _NSC * _NSUB
_CH = 128
_NCHK = _E // _CH
_CPT = _NCHK // _NW
_LEFT = _NCHK - _CPT * _NW
_RPT = 624

_mesh = plsc.VectorSubcoreMesh(core_axis_name="c", subcore_axis_name="s")


def _zero_my_slice(zer_v, acc_sh, sid):
    r0 = sid * _RPT

    @pl.loop(0, 13)
    def _(j):
        pltpu.sync_copy(zer_v, acc_sh.at[pl.ds(r0 + j * 48, 48)])

    @pl.when(sid == _NSUB - 1)
    def _():
        pltpu.sync_copy(zer_v.at[pl.ds(0, 16)], acc_sh.at[pl.ds(_N - 16, 16)])


def _write_my_slice(acc_sh, out_hbm, cid, sid):
    r0 = sid * _RPT

    @pl.loop(0, 13)
    def _(j):
        rr = r0 + j * 48
        pltpu.sync_copy(acc_sh.at[pl.ds(rr, 48)],
                        out_hbm.at[cid, pl.ds(rr, 48), :])

    @pl.when(sid == _NSUB - 1)
    def _():
        pltpu.sync_copy(acc_sh.at[pl.ds(_N - 16, 16)],
                        out_hbm.at[cid, pl.ds(_N - 16, 16), :])


@functools.partial(
    pl.kernel,
    out_type=jax.ShapeDtypeStruct((_NSC, _N, 16), jnp.float32),
    mesh=_mesh,
    scratch_types=[
        pltpu.VMEM((_CH,), jnp.int32),
        pltpu.VMEM((_CH,), jnp.int32),
        pltpu.VMEM((_CH,), jnp.int32),
        pltpu.VMEM((_CH,), jnp.int32),
        pltpu.VMEM((_CH, 16), jnp.float32),
        pltpu.VMEM((48, 16), jnp.float32),
        pltpu.VMEM_SHARED((_N, 16), jnp.float32),
        pltpu.SemaphoreType.DMA,
        pltpu.SemaphoreType.DMA,
        pltpu.SemaphoreType.DMA,
        pltpu.SemaphoreType.DMA,
        pltpu.SemaphoreType.DMA,
        pltpu.SemaphoreType.DMA,
        pltpu.SemaphoreType.DMA,
        pltpu.SemaphoreType.DMA,
    ],
)
def _deg_kernel(dst_hbm, out_hbm, di0, di1, di2, di3, ones_v, zer_v, acc_sh,
                semI0, semI1, semI2, semI3, semS0, semS1, semS2, semS3):
    cid = lax.axis_index("c")
    sid = lax.axis_index("s")
    wid = sid * _NSC + cid
    base = wid * _CPT

    dis = (di0, di1, di2, di3)
    semIs = (semI0, semI1, semI2, semI3)
    semSs = (semS0, semS1, semS2, semS3)

    def idx(c, k):
        return pltpu.make_async_copy(
            dst_hbm.at[pl.ds(c * _CH, _CH)], dis[k], semIs[k])

    def scat(k):
        return pltpu.make_async_copy(ones_v, acc_sh.at[dis[k]], semSs[k])

    idx(base, 0).start()
    idx(base + 1, 1).start()

    @pl.loop(0, 48)
    def _(i):
        zer_v[i, :] = jnp.zeros((16,), jnp.float32)

    @pl.loop(0, _CH)
    def _(i):
        ones_v[i, :] = jnp.ones((16,), jnp.float32)

    _zero_my_slice(zer_v, acc_sh, sid)
    plsc.subcore_barrier()

    def part(c, k, wait_prev2):
        idx(c, k).wait()
        scat(k).start(add=True)
        if wait_prev2:
            scat((k - 2) % 4).wait()

        @pl.when(c + 2 < base + _CPT)
        def _():
            idx(c + 2, (k + 2) % 4).start()

    part(base, 0, False)
    part(base + 1, 1, False)

    @pl.loop(0, (_CPT - 2) // 4)
    def _(u):
        c0 = base + 2 + u * 4
        part(c0, 2, True)
        part(c0 + 1, 3, True)
        part(c0 + 2, 0, True)
        part(c0 + 3, 1, True)

    scat(0).wait()
    scat(1).wait()

    @pl.when(wid < _LEFT)
    def _():
        c = _NW * _CPT + wid
        idx(c, 0).start()
        idx(c, 0).wait()
        scat(0).start(add=True)
        scat(0).wait()

    plsc.subcore_barrier()
    _write_my_slice(acc_sh, out_hbm, cid, sid)


@functools.partial(
    pl.kernel,
    out_type=jax.ShapeDtypeStruct((_NSC, _N, _D), jnp.float32),
    mesh=_mesh,
    scratch_types=[
        pltpu.VMEM((_CH,), jnp.int32),
        pltpu.VMEM((_CH,), jnp.int32),
        pltpu.VMEM((_CH,), jnp.int32),
        pltpu.VMEM((_CH,), jnp.int32),
        pltpu.VMEM((_CH,), jnp.int32),
        pltpu.VMEM((_CH,), jnp.int32),
        pltpu.VMEM((_CH, _D), jnp.float32),
        pltpu.VMEM((_CH, _D), jnp.float32),
        pltpu.VMEM((_CH, _D), jnp.float32),
        pltpu.VMEM_SHARED((_N, _D), jnp.float32),
        pltpu.SemaphoreType.DMA,
        pltpu.SemaphoreType.DMA,
        pltpu.SemaphoreType.DMA,
        pltpu.SemaphoreType.DMA,
        pltpu.SemaphoreType.DMA,
        pltpu.SemaphoreType.DMA,
        pltpu.SemaphoreType.DMA,
        pltpu.SemaphoreType.DMA,
        pltpu.SemaphoreType.DMA,
    ],
)
def _segsum_kernel(hs_hbm, src_hbm, dst_hbm, out_hbm,
                   si0, si1, si2, di0, di1, di2,
                   r0_v, r1_v, r2_v, acc_sh,
                   semI0, semI1, semI2,
                   semG0, semG1, semG2,
                   semS0, semS1, semS2):
    cid = lax.axis_index("c")
    sid = lax.axis_index("s")
    wid = sid * _NSC + cid
    base = wid * _CPT

    sis = (si0, si1, si2)
    dis = (di0, di1, di2)
    rows = (r0_v, r1_v, r2_v)
    semIs = (semI0, semI1, semI2)
    semGs = (semG0, semG1, semG2)
    semSs = (semS0, semS1, semS2)

    def idx(c, k):
        return (pltpu.make_async_copy(
                    dst_hbm.at[pl.ds(c * _CH, _CH)], dis[k], semIs[k]),
                pltpu.make_async_copy(
                    src_hbm.at[pl.ds(c * _CH, _CH)], sis[k], semIs[k]))

    def idx_start(c, k):
        a, b = idx(c, k)
        a.start()
        b.start()

    def idx_wait(c, k):
        a, b = idx(c, k)
        a.wait()
        b.wait()

    def gat(k):
        return pltpu.make_async_copy(hs_hbm.at[sis[k]], rows[k], semGs[k])

    def scat(k):
        return pltpu.make_async_copy(rows[k], acc_sh.at[dis[k]], semSs[k])

    idx_start(base, 0)
    idx_start(base + 1, 1)

    @pl.loop(0, _CH)
    def _(i):
        @pl.loop(0, _D // 16)
        def _(j):
            r0_v[i, pl.ds(j * 16, 16)] = jnp.zeros((16,), jnp.float32)

    rbase = sid * _RPT

    @pl.loop(0, 4)
    def _(j):
        pltpu.sync_copy(r0_v, acc_sh.at[pl.ds(rbase + j * _CH, _CH)])
    pltpu.sync_copy(r0_v.at[pl.ds(0, 112)],
                    acc_sh.at[pl.ds(rbase + 4 * _CH, 112)])

    @pl.when(sid == _NSUB - 1)
    def _():
        pltpu.sync_copy(r0_v.at[pl.ds(0, 16)], acc_sh.at[pl.ds(_N - 16, 16)])

    idx_wait(base, 0)
    gat(0).start()
    plsc.subcore_barrier()

    def part(c, k, wait_prev):
        gat(k).wait()

        @pl.when(c + 1 < base + _CPT)
        def _():
            idx_wait(c + 1, (k + 1) % 3)
            gat((k + 1) % 3).start()

        scat(k).start(add=True)
        if wait_prev:
            scat((k - 1) % 3).wait()

        @pl.when(c + 2 < base + _CPT)
        def _():
            idx_start(c + 2, (k + 2) % 3)

    part(base, 0, False)
    part(base + 1, 1, True)
    part(base + 2, 2, True)

    @pl.loop(0, (_CPT - 3) // 3)
    def _(u):
        c0 = base + 3 + u * 3
        part(c0, 0, True)
        part(c0 + 1, 1, True)
        part(c0 + 2, 2, True)

    scat((_CPT - 1) % 3).wait()

    @pl.when(wid < _LEFT)
    def _():
        c = _NW * _CPT + wid
        idx_start(c, 0)
        idx_wait(c, 0)
        gat(0).start()
        gat(0).wait()
        scat(0).start(add=True)
        scat(0).wait()

    plsc.subcore_barrier()
    _write_my_slice(acc_sh, out_hbm, cid, sid)


_BLK = 1000


def _tc1_body(x_ref, lx_ref, w_ref, b_ref, xo_ref, h_ref):
    xo = x_ref[...] * lx_ref[...]
    xo_ref[...] = xo
    h_ref[...] = (jnp.dot(xo, w_ref[...], preferred_element_type=jnp.float32)
                  + b_ref[...])


def _tc2_body(d0_ref, d1_ref, h_ref, hs_ref):
    dinv = lax.rsqrt(d0_ref[...] + d1_ref[...] + 1.0)
    hs_ref[...] = h_ref[...] * dinv


def _tc3_body(d0_ref, d1_ref, s0_ref, s1_ref, hs_ref, w_ref, b_ref, out_ref):
    dinv = lax.rsqrt(d0_ref[...] + d1_ref[...] + 1.0)
    t = (s0_ref[...] + s1_ref[...] + hs_ref[...]) * dinv
    h1 = jnp.maximum(t, 0.0)
    out_ref[...] = (jnp.dot(h1, w_ref[...], preferred_element_type=jnp.float32)
                    + b_ref[...]) * dinv


def _tc4_body(d0_ref, d1_ref, s0_ref, s1_ref, hs_ref,
              w1_ref, b1_ref, w2_ref, b2_ref, out_ref):
    dinv = lax.rsqrt(d0_ref[...] + d1_ref[...] + 1.0)
    embed = (s0_ref[...] + s1_ref[...] + hs_ref[...]) * dinv
    hidden = jnp.maximum(
        jnp.dot(embed, w1_ref[...], preferred_element_type=jnp.float32)
        + b1_ref[...], 0.0)
    out_ref[...] = (jnp.dot(hidden, w2_ref[...],
                            preferred_element_type=jnp.float32) + b2_ref[...])


def _row_spec():
    return pl.BlockSpec((_BLK, _D), lambda i: (i, 0))


def _deg_spec():
    return pl.BlockSpec((_BLK, 1), lambda i: (i, 0))


def _full_spec(shape):
    return pl.BlockSpec(shape, lambda i: tuple(0 for _ in shape))


def kernel(x, edge_index, emb1, emb3, learnable_x, cond_Wi, cond_bi, cond_Wo,
           cond_bo, g_W1, g_b1, g_W2, g_b2, c_W1, c_b1, c_W2, c_b2):
    src = edge_index[0]
    dst = edge_index[1]
    n, d = x.shape
    grid = (n // _BLK,)

    degp = _deg_kernel(dst)
    d0 = degp[0, :, 0].reshape(n, 1)
    d1 = degp[1, :, 0].reshape(n, 1)

    x_out, h1_pre = pl.pallas_call(
        _tc1_body,
        grid=grid,
        in_specs=[_row_spec(), _row_spec(),
                  _full_spec((_D, _D)), _full_spec((1, _D))],
        out_specs=[_row_spec(), _row_spec()],
        out_shape=[jax.ShapeDtypeStruct((n, d), jnp.float32)] * 2,
    )(x, learnable_x, g_W1, g_b1.reshape(1, d))

    hs1 = pl.pallas_call(
        _tc2_body,
        grid=grid,
        in_specs=[_deg_spec(), _deg_spec(), _row_spec()],
        out_specs=_row_spec(),
        out_shape=jax.ShapeDtypeStruct((n, d), jnp.float32),
    )(d0, d1, h1_pre)

    s1 = _segsum_kernel(hs1, src, dst)

    hs2 = pl.pallas_call(
        _tc3_body,
        grid=grid,
        in_specs=[_deg_spec(), _deg_spec(), _row_spec(), _row_spec(),
                  _row_spec(), _full_spec((_D, _D)), _full_spec((1, _D))],
        out_specs=_row_spec(),
        out_shape=jax.ShapeDtypeStruct((n, d), jnp.float32),
    )(d0, d1, s1[0], s1[1], hs1, g_W2, g_b2.reshape(1, d))

    s2 = _segsum_kernel(hs2, src, dst)

    nh = c_W1.shape[1]
    nc = c_W2.shape[1]
    logits = pl.pallas_call(
        _tc4_body,
        grid=grid,
        in_specs=[_deg_spec(), _deg_spec(), _row_spec(), _row_spec(),
                  _row_spec(), _full_spec((_D, nh)), _full_spec((1, nh)),
                  _full_spec((nh, nc)), _full_spec((1, nc))],
        out_specs=pl.BlockSpec((_BLK, nc), lambda i: (i, 0)),
        out_shape=jax.ShapeDtypeStruct((n, nc), jnp.float32),
    )(d0, d1, s2[0], s2[1], hs2, c_W1, c_b1.reshape(1, nh),
      c_W2, c_b2.reshape(1, nc))

    return (x_out, logits)


# speedup vs baseline: 1.0847x; 1.0656x over previous
import functools

import jax
import jax.numpy as jnp
from jax import lax
from jax.experimental import pallas as pl
from jax.experimental.pallas import tpu as pltpu
from jax.experimental.pallas import tpu_sc as plsc

_N = 10000
_E = 320000
_D = 128

_NSC = 2
_NSUB = 16
_NW = _NSC * _NSUB
_CH = 80
_NCHK = _E // _CH
_CPT = _NCHK // _NW
_LEFT = _NCHK - _CPT * _NW
_RPT = 624

_mesh = plsc.VectorSubcoreMesh(core_axis_name="c", subcore_axis_name="s")


def _zero_my_slice(zer_v, acc_sh, sid):
    r0 = sid * _RPT

    @pl.loop(0, 13)
    def _(j):
        pltpu.sync_copy(zer_v, acc_sh.at[pl.ds(r0 + j * 48, 48)])

    @pl.when(sid == _NSUB - 1)
    def _():
        pltpu.sync_copy(zer_v.at[pl.ds(0, 16)], acc_sh.at[pl.ds(_N - 16, 16)])


def _write_my_slice(acc_sh, out_hbm, cid, sid):
    r0 = sid * _RPT

    @pl.loop(0, 13)
    def _(j):
        rr = r0 + j * 48
        pltpu.sync_copy(acc_sh.at[pl.ds(rr, 48)],
                        out_hbm.at[cid, pl.ds(rr, 48), :])

    @pl.when(sid == _NSUB - 1)
    def _():
        pltpu.sync_copy(acc_sh.at[pl.ds(_N - 16, 16)],
                        out_hbm.at[cid, pl.ds(_N - 16, 16), :])


@functools.partial(
    pl.kernel,
    out_type=jax.ShapeDtypeStruct((_NSC, _N, 16), jnp.float32),
    mesh=_mesh,
    scratch_types=[
        pltpu.VMEM((_CH,), jnp.int32),
        pltpu.VMEM((_CH,), jnp.int32),
        pltpu.VMEM((_CH,), jnp.int32),
        pltpu.VMEM((_CH,), jnp.int32),
        pltpu.VMEM((_CH, 16), jnp.float32),
        pltpu.VMEM((48, 16), jnp.float32),
        pltpu.VMEM_SHARED((_N, 16), jnp.float32),
        pltpu.SemaphoreType.DMA,
        pltpu.SemaphoreType.DMA,
        pltpu.SemaphoreType.DMA,
        pltpu.SemaphoreType.DMA,
        pltpu.SemaphoreType.DMA,
        pltpu.SemaphoreType.DMA,
        pltpu.SemaphoreType.DMA,
        pltpu.SemaphoreType.DMA,
    ],
)
def _deg_kernel(dst_hbm, out_hbm, di0, di1, di2, di3, ones_v, zer_v, acc_sh,
                semI0, semI1, semI2, semI3, semS0, semS1, semS2, semS3):
    cid = lax.axis_index("c")
    sid = lax.axis_index("s")
    wid = sid * _NSC + cid
    base = wid * _CPT

    dis = (di0, di1, di2, di3)
    semIs = (semI0, semI1, semI2, semI3)
    semSs = (semS0, semS1, semS2, semS3)

    def idx(c, k):
        return pltpu.make_async_copy(
            dst_hbm.at[pl.ds(c * _CH, _CH)], dis[k], semIs[k])

    def scat(k):
        return pltpu.make_async_copy(ones_v, acc_sh.at[dis[k]], semSs[k])

    idx(base, 0).start()
    idx(base + 1, 1).start()

    @pl.loop(0, 48)
    def _(i):
        zer_v[i, :] = jnp.zeros((16,), jnp.float32)

    @pl.loop(0, _CH)
    def _(i):
        ones_v[i, :] = jnp.ones((16,), jnp.float32)

    _zero_my_slice(zer_v, acc_sh, sid)
    plsc.subcore_barrier()

    def part(c, k, wait_prev2):
        idx(c, k).wait()
        scat(k).start(add=True)
        if wait_prev2:
            scat((k - 2) % 4).wait()

        @pl.when(c + 2 < base + _CPT)
        def _():
            idx(c + 2, (k + 2) % 4).start()

    part(base, 0, False)
    part(base + 1, 1, False)

    @pl.loop(0, (_CPT - 5) // 4)
    def _(u):
        c0 = base + 2 + u * 4
        part(c0, 2, True)
        part(c0 + 1, 3, True)
        part(c0 + 2, 0, True)
        part(c0 + 3, 1, True)

    part(base + _CPT - 3, 2, True)
    part(base + _CPT - 2, 3, True)
    part(base + _CPT - 1, 0, True)

    scat(3).wait()
    scat(0).wait()

    plsc.subcore_barrier()
    _write_my_slice(acc_sh, out_hbm, cid, sid)


@functools.partial(
    pl.kernel,
    out_type=jax.ShapeDtypeStruct((_NSC, _N, _D), jnp.float32),
    mesh=_mesh,
    scratch_types=[
        pltpu.VMEM((_CH,), jnp.int32),
        pltpu.VMEM((_CH,), jnp.int32),
        pltpu.VMEM((_CH,), jnp.int32),
        pltpu.VMEM((_CH,), jnp.int32),
        pltpu.VMEM((_CH,), jnp.int32),
        pltpu.VMEM((_CH,), jnp.int32),
        pltpu.VMEM((_CH,), jnp.int32),
        pltpu.VMEM((_CH,), jnp.int32),
        pltpu.VMEM((_CH, _D), jnp.float32),
        pltpu.VMEM((_CH, _D), jnp.float32),
        pltpu.VMEM((_CH, _D), jnp.float32),
        pltpu.VMEM((_CH, _D), jnp.float32),
        pltpu.VMEM_SHARED((_N, _D), jnp.float32),
        pltpu.SemaphoreType.DMA,
        pltpu.SemaphoreType.DMA,
        pltpu.SemaphoreType.DMA,
        pltpu.SemaphoreType.DMA,
        pltpu.SemaphoreType.DMA,
        pltpu.SemaphoreType.DMA,
        pltpu.SemaphoreType.DMA,
        pltpu.SemaphoreType.DMA,
        pltpu.SemaphoreType.DMA,
        pltpu.SemaphoreType.DMA,
        pltpu.SemaphoreType.DMA,
        pltpu.SemaphoreType.DMA,
        pltpu.SemaphoreType.DMA,
        pltpu.SemaphoreType.DMA,
        pltpu.SemaphoreType.DMA,
        pltpu.SemaphoreType.DMA,
    ],
)
def _segsum_kernel(hs_hbm, src_hbm, dst_hbm, out_hbm,
                   si0, si1, si2, si3, di0, di1, di2, di3,
                   r0_v, r1_v, r2_v, r3_v, acc_sh,
                   semIS0, semIS1, semIS2, semIS3,
                   semID0, semID1, semID2, semID3,
                   semG0, semG1, semG2, semG3,
                   semS0, semS1, semS2, semS3):
    cid = lax.axis_index("c")
    sid = lax.axis_index("s")
    wid = sid * _NSC + cid
    base = wid * _CPT
    end = base + _CPT

    sis = (si0, si1, si2, si3)
    dis = (di0, di1, di2, di3)
    rows = (r0_v, r1_v, r2_v, r3_v)
    semISs = (semIS0, semIS1, semIS2, semIS3)
    semIDs = (semID0, semID1, semID2, semID3)
    semGs = (semG0, semG1, semG2, semG3)
    semSs = (semS0, semS1, semS2, semS3)

    def isrc_k(c, k):
        return pltpu.make_async_copy(
            src_hbm.at[pl.ds(c * _CH, _CH)], sis[k], semISs[k])

    def idst_k(c, k):
        return pltpu.make_async_copy(
            dst_hbm.at[pl.ds(c * _CH, _CH)], dis[k], semIDs[k])

    def gat(k):
        return pltpu.make_async_copy(hs_hbm.at[sis[k]], rows[k], semGs[k])

    def scat(k):
        return pltpu.make_async_copy(rows[k], acc_sh.at[dis[k]], semSs[k])

    isrc_k(base, 0).start()
    idst_k(base, 0).start()
    isrc_k(base + 1, 1).start()
    idst_k(base + 1, 1).start()

    @pl.loop(0, _CH)
    def _(i):
        @pl.loop(0, _D // 16)
        def _(j):
            r3_v[i, pl.ds(j * 16, 16)] = jnp.zeros((16,), jnp.float32)

    rbase = sid * _RPT

    @pl.loop(0, 7)
    def _(j):
        pltpu.sync_copy(r3_v, acc_sh.at[pl.ds(rbase + j * _CH, _CH)])
    pltpu.sync_copy(r3_v.at[pl.ds(0, 64)],
                    acc_sh.at[pl.ds(rbase + 7 * _CH, 64)])

    @pl.when(sid == _NSUB - 1)
    def _():
        pltpu.sync_copy(r3_v.at[pl.ds(0, 16)], acc_sh.at[pl.ds(_N - 16, 16)])

    isrc_k(base, 0).wait()
    gat(0).start()
    isrc_k(base + 1, 1).wait()
    gat(1).start()
    isrc_k(base + 2, 2).start()
    isrc_k(base + 3, 3).start()
    plsc.subcore_barrier()

    def part(c, k, first, s4, w2):
        gat(k).wait()
        if not first:
            scat((k + 2) % 4).wait()
        if w2:
            @pl.when(c + 2 < end)
            def _():
                idst_k(c + 2, (k + 2) % 4).start()
                isrc_k(c + 2, (k + 2) % 4).wait()
                gat((k + 2) % 4).start()
        if s4:
            @pl.when(c + 4 < end)
            def _():
                isrc_k(c + 4, k).start()
        idst_k(c, k).wait()
        scat(k).start(add=True)

    part(base + 0, 0, True, True, True)
    part(base + 1, 1, True, True, True)

    @pl.loop(0, (_CPT - 5) // 4)
    def _(u):
        c0 = base + 2 + u * 4
        part(c0, 2, False, True, True)
        part(c0 + 1, 3, False, True, True)
        part(c0 + 2, 0, False, True, True)
        part(c0 + 3, 1, False, True, True)

    part(end - 3, 2, False, False, True)
    part(end - 2, 3, False, False, False)
    part(end - 1, 0, False, False, False)

    scat(3).wait()
    scat(0).wait()

    plsc.subcore_barrier()
    _write_my_slice(acc_sh, out_hbm, cid, sid)


_BLK = 1000


def _tc1_body(x_ref, lx_ref, w_ref, b_ref, xo_ref, h_ref):
    xo = x_ref[...] * lx_ref[...]
    xo_ref[...] = xo
    h_ref[...] = (jnp.dot(xo, w_ref[...], preferred_element_type=jnp.float32)
                  + b_ref[...])


def _tc2_body(d0_ref, d1_ref, h_ref, hs_ref):
    dinv = lax.rsqrt(d0_ref[...] + d1_ref[...] + 1.0)
    hs_ref[...] = h_ref[...] * dinv


def _tc3_body(d0_ref, d1_ref, s0_ref, s1_ref, hs_ref, w_ref, b_ref, out_ref):
    dinv = lax.rsqrt(d0_ref[...] + d1_ref[...] + 1.0)
    t = (s0_ref[...] + s1_ref[...] + hs_ref[...]) * dinv
    h1 = jnp.maximum(t, 0.0)
    out_ref[...] = (jnp.dot(h1, w_ref[...], preferred_element_type=jnp.float32)
                    + b_ref[...]) * dinv


def _tc4_body(d0_ref, d1_ref, s0_ref, s1_ref, hs_ref,
              w1_ref, b1_ref, w2_ref, b2_ref, out_ref):
    dinv = lax.rsqrt(d0_ref[...] + d1_ref[...] + 1.0)
    embed = (s0_ref[...] + s1_ref[...] + hs_ref[...]) * dinv
    hidden = jnp.maximum(
        jnp.dot(embed, w1_ref[...], preferred_element_type=jnp.float32)
        + b1_ref[...], 0.0)
    out_ref[...] = (jnp.dot(hidden, w2_ref[...],
                            preferred_element_type=jnp.float32) + b2_ref[...])


def _row_spec():
    return pl.BlockSpec((_BLK, _D), lambda i: (i, 0))


def _deg_spec():
    return pl.BlockSpec((_BLK, 1), lambda i: (i, 0))


def _full_spec(shape):
    return pl.BlockSpec(shape, lambda i: tuple(0 for _ in shape))


def kernel(x, edge_index, emb1, emb3, learnable_x, cond_Wi, cond_bi, cond_Wo,
           cond_bo, g_W1, g_b1, g_W2, g_b2, c_W1, c_b1, c_W2, c_b2):
    src = edge_index[0]
    dst = edge_index[1]
    n, d = x.shape
    grid = (n // _BLK,)

    degp = _deg_kernel(dst)
    d0 = degp[0, :, 0].reshape(n, 1)
    d1 = degp[1, :, 0].reshape(n, 1)

    x_out, h1_pre = pl.pallas_call(
        _tc1_body,
        grid=grid,
        in_specs=[_row_spec(), _row_spec(),
                  _full_spec((_D, _D)), _full_spec((1, _D))],
        out_specs=[_row_spec(), _row_spec()],
        out_shape=[jax.ShapeDtypeStruct((n, d), jnp.float32)] * 2,
    )(x, learnable_x, g_W1, g_b1.reshape(1, d))

    hs1 = pl.pallas_call(
        _tc2_body,
        grid=grid,
        in_specs=[_deg_spec(), _deg_spec(), _row_spec()],
        out_specs=_row_spec(),
        out_shape=jax.ShapeDtypeStruct((n, d), jnp.float32),
    )(d0, d1, h1_pre)

    s1 = _segsum_kernel(hs1, src, dst)

    hs2 = pl.pallas_call(
        _tc3_body,
        grid=grid,
        in_specs=[_deg_spec(), _deg_spec(), _row_spec(), _row_spec(),
                  _row_spec(), _full_spec((_D, _D)), _full_spec((1, _D))],
        out_specs=_row_spec(),
        out_shape=jax.ShapeDtypeStruct((n, d), jnp.float32),
    )(d0, d1, s1[0], s1[1], hs1, g_W2, g_b2.reshape(1, d))

    s2 = _segsum_kernel(hs2, src, dst)

    nh = c_W1.shape[1]
    nc = c_W2.shape[1]
    logits = pl.pallas_call(
        _tc4_body,
        grid=grid,
        in_specs=[_deg_spec(), _deg_spec(), _row_spec(), _row_spec(),
                  _row_spec(), _full_spec((_D, nh)), _full_spec((1, nh)),
                  _full_spec((nh, nc)), _full_spec((1, nc))],
        out_specs=pl.BlockSpec((_BLK, nc), lambda i: (i, 0)),
        out_shape=jax.ShapeDtypeStruct((n, nc), jnp.float32),
    )(d0, d1, s2[0], s2[1], hs2, c_W1, c_b1.reshape(1, nh),
      c_W2, c_b2.reshape(1, nc))

    return (x_out, logits)


# speedup vs baseline: 1.1116x; 1.0249x over previous
import functools

import jax
import jax.numpy as jnp
from jax import lax
from jax.experimental import pallas as pl
from jax.experimental.pallas import tpu as pltpu
from jax.experimental.pallas import tpu_sc as plsc

_N = 10000
_E = 320000
_D = 128

_NSC = 2
_NSUB = 16
_NW = _NSC * _NSUB
_CH = 80
_NCHK = _E // _CH
_CPT = _NCHK // _NW
_LEFT = _NCHK - _CPT * _NW
_DCH = 128
_DCPT = (_E // _DCH) // _NW
_DLEFT = _E // _DCH - _DCPT * _NW
_RPT = 624

_mesh = plsc.VectorSubcoreMesh(core_axis_name="c", subcore_axis_name="s")


def _zero_my_slice(zer_v, acc_sh, sid, sem):
    r0 = sid * _RPT
    cs = [pltpu.make_async_copy(zer_v, acc_sh.at[pl.ds(r0 + j * 48, 48)], sem)
          for j in range(13)]
    for c in cs:
        c.start()

    @pl.when(sid == _NSUB - 1)
    def _():
        pltpu.sync_copy(zer_v.at[pl.ds(0, 16)], acc_sh.at[pl.ds(_N - 16, 16)])

    for c in cs:
        c.wait()


def _write_my_slice(acc_sh, out_hbm, cid, sid):
    r0 = sid * _RPT

    @pl.loop(0, 13)
    def _(j):
        rr = r0 + j * 48
        pltpu.sync_copy(acc_sh.at[pl.ds(rr, 48)],
                        out_hbm.at[cid, pl.ds(rr, 48), :])

    @pl.when(sid == _NSUB - 1)
    def _():
        pltpu.sync_copy(acc_sh.at[pl.ds(_N - 16, 16)],
                        out_hbm.at[cid, pl.ds(_N - 16, 16), :])


@functools.partial(
    pl.kernel,
    out_type=jax.ShapeDtypeStruct((_NSC, _N, 16), jnp.float32),
    mesh=_mesh,
    scratch_types=[
        pltpu.VMEM((_DCH,), jnp.int32),
        pltpu.VMEM((_DCH,), jnp.int32),
        pltpu.VMEM((_DCH,), jnp.int32),
        pltpu.VMEM((_DCH,), jnp.int32),
        pltpu.VMEM((_DCH, 16), jnp.float32),
        pltpu.VMEM((48, 16), jnp.float32),
        pltpu.VMEM_SHARED((_N, 16), jnp.float32),
        pltpu.SemaphoreType.DMA,
        pltpu.SemaphoreType.DMA,
        pltpu.SemaphoreType.DMA,
        pltpu.SemaphoreType.DMA,
        pltpu.SemaphoreType.DMA,
        pltpu.SemaphoreType.DMA,
        pltpu.SemaphoreType.DMA,
        pltpu.SemaphoreType.DMA,
    ],
)
def _deg_kernel(dst_hbm, out_hbm, di0, di1, di2, di3, ones_v, zer_v, acc_sh,
                semI0, semI1, semI2, semI3, semS0, semS1, semS2, semS3):
    cid = lax.axis_index("c")
    sid = lax.axis_index("s")
    wid = sid * _NSC + cid
    base = wid * _DCPT

    dis = (di0, di1, di2, di3)
    semIs = (semI0, semI1, semI2, semI3)
    semSs = (semS0, semS1, semS2, semS3)

    def idx(c, k):
        return pltpu.make_async_copy(
            dst_hbm.at[pl.ds(c * _DCH, _DCH)], dis[k], semIs[k])

    def scat(k):
        return pltpu.make_async_copy(ones_v, acc_sh.at[dis[k]], semSs[k])

    idx(base, 0).start()
    idx(base + 1, 1).start()

    @pl.loop(0, 48)
    def _(i):
        zer_v[i, :] = jnp.zeros((16,), jnp.float32)

    @pl.loop(0, _DCH)
    def _(i):
        ones_v[i, :] = jnp.ones((16,), jnp.float32)

    _zero_my_slice(zer_v, acc_sh, sid, semS3)
    plsc.subcore_barrier()

    def part(c, k, wait_prev2):
        idx(c, k).wait()
        scat(k).start(add=True)
        if wait_prev2:
            scat((k - 2) % 4).wait()

        @pl.when(c + 2 < base + _DCPT)
        def _():
            idx(c + 2, (k + 2) % 4).start()

    part(base, 0, False)
    part(base + 1, 1, False)

    @pl.loop(0, (_DCPT - 2) // 4)
    def _(u):
        c0 = base + 2 + u * 4
        part(c0, 2, True)
        part(c0 + 1, 3, True)
        part(c0 + 2, 0, True)
        part(c0 + 3, 1, True)

    scat(0).wait()
    scat(1).wait()

    @pl.when(wid < _DLEFT)
    def _():
        c = _NW * _DCPT + wid
        idx(c, 0).start()
        idx(c, 0).wait()
        scat(0).start(add=True)
        scat(0).wait()

    plsc.subcore_barrier()
    _write_my_slice(acc_sh, out_hbm, cid, sid)


@functools.partial(
    pl.kernel,
    out_type=jax.ShapeDtypeStruct((_NSC, _N, _D), jnp.float32),
    mesh=_mesh,
    scratch_types=[
        pltpu.VMEM((_CH,), jnp.int32),
        pltpu.VMEM((_CH,), jnp.int32),
        pltpu.VMEM((_CH,), jnp.int32),
        pltpu.VMEM((_CH,), jnp.int32),
        pltpu.VMEM((_CH,), jnp.int32),
        pltpu.VMEM((_CH,), jnp.int32),
        pltpu.VMEM((_CH,), jnp.int32),
        pltpu.VMEM((_CH,), jnp.int32),
        pltpu.VMEM((_CH, _D), jnp.float32),
        pltpu.VMEM((_CH, _D), jnp.float32),
        pltpu.VMEM((_CH, _D), jnp.float32),
        pltpu.VMEM((_CH, _D), jnp.float32),
        pltpu.VMEM_SHARED((_N, _D), jnp.float32),
        pltpu.SemaphoreType.DMA,
        pltpu.SemaphoreType.DMA,
        pltpu.SemaphoreType.DMA,
        pltpu.SemaphoreType.DMA,
        pltpu.SemaphoreType.DMA,
        pltpu.SemaphoreType.DMA,
        pltpu.SemaphoreType.DMA,
        pltpu.SemaphoreType.DMA,
        pltpu.SemaphoreType.DMA,
        pltpu.SemaphoreType.DMA,
        pltpu.SemaphoreType.DMA,
        pltpu.SemaphoreType.DMA,
        pltpu.SemaphoreType.DMA,
        pltpu.SemaphoreType.DMA,
        pltpu.SemaphoreType.DMA,
        pltpu.SemaphoreType.DMA,
    ],
)
def _segsum_kernel(hs_hbm, src_hbm, dst_hbm, out_hbm,
                   si0, si1, si2, si3, di0, di1, di2, di3,
                   r0_v, r1_v, r2_v, r3_v, acc_sh,
                   semIS0, semIS1, semIS2, semIS3,
                   semID0, semID1, semID2, semID3,
                   semG0, semG1, semG2, semG3,
                   semS0, semS1, semS2, semS3):
    cid = lax.axis_index("c")
    sid = lax.axis_index("s")
    wid = sid * _NSC + cid
    base = wid * _CPT
    end = base + _CPT

    sis = (si0, si1, si2, si3)
    dis = (di0, di1, di2, di3)
    rows = (r0_v, r1_v, r2_v, r3_v)
    semISs = (semIS0, semIS1, semIS2, semIS3)
    semIDs = (semID0, semID1, semID2, semID3)
    semGs = (semG0, semG1, semG2, semG3)
    semSs = (semS0, semS1, semS2, semS3)

    def isrc_k(c, k):
        return pltpu.make_async_copy(
            src_hbm.at[pl.ds(c * _CH, _CH)], sis[k], semISs[k])

    def idst_k(c, k):
        return pltpu.make_async_copy(
            dst_hbm.at[pl.ds(c * _CH, _CH)], dis[k], semIDs[k])

    def gat(k):
        return pltpu.make_async_copy(hs_hbm.at[sis[k]], rows[k], semGs[k])

    def scat(k):
        return pltpu.make_async_copy(rows[k], acc_sh.at[dis[k]], semSs[k])

    isrc_k(base, 0).start()
    idst_k(base, 0).start()
    isrc_k(base + 1, 1).start()
    idst_k(base + 1, 1).start()

    @pl.loop(0, _CH)
    def _(i):
        @pl.loop(0, _D // 16)
        def _(j):
            r3_v[i, pl.ds(j * 16, 16)] = jnp.zeros((16,), jnp.float32)

    rbase = sid * _RPT
    zcs = [pltpu.make_async_copy(
               r3_v, acc_sh.at[pl.ds(rbase + j * _CH, _CH)], semS3)
           for j in range(7)]
    zcs.append(pltpu.make_async_copy(
        r3_v.at[pl.ds(0, 64)], acc_sh.at[pl.ds(rbase + 7 * _CH, 64)], semS3))
    for z in zcs:
        z.start()

    @pl.when(sid == _NSUB - 1)
    def _():
        pltpu.sync_copy(r3_v.at[pl.ds(0, 16)], acc_sh.at[pl.ds(_N - 16, 16)])

    for z in zcs:
        z.wait()

    isrc_k(base, 0).wait()
    gat(0).start()
    isrc_k(base + 1, 1).wait()
    gat(1).start()
    isrc_k(base + 2, 2).start()
    isrc_k(base + 3, 3).start()
    plsc.subcore_barrier()

    def part(c, k, first, s4, w2):
        gat(k).wait()
        if not first:
            scat((k + 2) % 4).wait()
        if w2:
            @pl.when(c + 2 < end)
            def _():
                idst_k(c + 2, (k + 2) % 4).start()
                isrc_k(c + 2, (k + 2) % 4).wait()
                gat((k + 2) % 4).start()
        if s4:
            @pl.when(c + 4 < end)
            def _():
                isrc_k(c + 4, k).start()
        idst_k(c, k).wait()
        scat(k).start(add=True)

    part(base + 0, 0, True, True, True)
    part(base + 1, 1, True, True, True)

    @pl.loop(0, (_CPT - 5) // 4)
    def _(u):
        c0 = base + 2 + u * 4
        part(c0, 2, False, True, True)
        part(c0 + 1, 3, False, True, True)
        part(c0 + 2, 0, False, True, True)
        part(c0 + 3, 1, False, True, True)

    part(end - 3, 2, False, False, True)
    part(end - 2, 3, False, False, False)
    part(end - 1, 0, False, False, False)

    scat(3).wait()
    scat(0).wait()

    plsc.subcore_barrier()
    _write_my_slice(acc_sh, out_hbm, cid, sid)


_BLK = 1000


def _tc1_body(x_ref, lx_ref, w_ref, b_ref, xo_ref, h_ref):
    xo = x_ref[...] * lx_ref[...]
    xo_ref[...] = xo
    h_ref[...] = (jnp.dot(xo, w_ref[...], preferred_element_type=jnp.float32)
                  + b_ref[...])


def _tc2_body(d0_ref, d1_ref, h_ref, hs_ref):
    dinv = lax.rsqrt(d0_ref[...] + d1_ref[...] + 1.0)
    hs_ref[...] = h_ref[...] * dinv


def _tc3_body(d0_ref, d1_ref, s0_ref, s1_ref, hs_ref, w_ref, b_ref, out_ref):
    dinv = lax.rsqrt(d0_ref[...] + d1_ref[...] + 1.0)
    t = (s0_ref[...] + s1_ref[...] + hs_ref[...]) * dinv
    h1 = jnp.maximum(t, 0.0)
    out_ref[...] = (jnp.dot(h1, w_ref[...], preferred_element_type=jnp.float32)
                    + b_ref[...]) * dinv


def _tc4_body(d0_ref, d1_ref, s0_ref, s1_ref, hs_ref,
              w1_ref, b1_ref, w2_ref, b2_ref, out_ref):
    dinv = lax.rsqrt(d0_ref[...] + d1_ref[...] + 1.0)
    embed = (s0_ref[...] + s1_ref[...] + hs_ref[...]) * dinv
    hidden = jnp.maximum(
        jnp.dot(embed, w1_ref[...], preferred_element_type=jnp.float32)
        + b1_ref[...], 0.0)
    out_ref[...] = (jnp.dot(hidden, w2_ref[...],
                            preferred_element_type=jnp.float32) + b2_ref[...])


def _row_spec():
    return pl.BlockSpec((_BLK, _D), lambda i: (i, 0))


def _deg_spec():
    return pl.BlockSpec((_BLK, 1), lambda i: (i, 0))


def _full_spec(shape):
    return pl.BlockSpec(shape, lambda i: tuple(0 for _ in shape))


def kernel(x, edge_index, emb1, emb3, learnable_x, cond_Wi, cond_bi, cond_Wo,
           cond_bo, g_W1, g_b1, g_W2, g_b2, c_W1, c_b1, c_W2, c_b2):
    src = edge_index[0]
    dst = edge_index[1]
    n, d = x.shape
    grid = (n // _BLK,)

    degp = _deg_kernel(dst)
    d0 = degp[0, :, 0].reshape(n, 1)
    d1 = degp[1, :, 0].reshape(n, 1)

    x_out, h1_pre = pl.pallas_call(
        _tc1_body,
        grid=grid,
        in_specs=[_row_spec(), _row_spec(),
                  _full_spec((_D, _D)), _full_spec((1, _D))],
        out_specs=[_row_spec(), _row_spec()],
        out_shape=[jax.ShapeDtypeStruct((n, d), jnp.float32)] * 2,
    )(x, learnable_x, g_W1, g_b1.reshape(1, d))

    hs1 = pl.pallas_call(
        _tc2_body,
        grid=grid,
        in_specs=[_deg_spec(), _deg_spec(), _row_spec()],
        out_specs=_row_spec(),
        out_shape=jax.ShapeDtypeStruct((n, d), jnp.float32),
    )(d0, d1, h1_pre)

    s1 = _segsum_kernel(hs1, src, dst)

    hs2 = pl.pallas_call(
        _tc3_body,
        grid=grid,
        in_specs=[_deg_spec(), _deg_spec(), _row_spec(), _row_spec(),
                  _row_spec(), _full_spec((_D, _D)), _full_spec((1, _D))],
        out_specs=_row_spec(),
        out_shape=jax.ShapeDtypeStruct((n, d), jnp.float32),
    )(d0, d1, s1[0], s1[1], hs1, g_W2, g_b2.reshape(1, d))

    s2 = _segsum_kernel(hs2, src, dst)

    nh = c_W1.shape[1]
    nc = c_W2.shape[1]
    logits = pl.pallas_call(
        _tc4_body,
        grid=grid,
        in_specs=[_deg_spec(), _deg_spec(), _row_spec(), _row_spec(),
                  _row_spec(), _full_spec((_D, nh)), _full_spec((1, nh)),
                  _full_spec((nh, nc)), _full_spec((1, nc))],
        out_specs=pl.BlockSpec((_BLK, nc), lambda i: (i, 0)),
        out_shape=jax.ShapeDtypeStruct((n, nc), jnp.float32),
    )(d0, d1, s2[0], s2[1], hs2, c_W1, c_b1.reshape(1, nh),
      c_W2, c_b2.reshape(1, nc))

    return (x_out, logits)


# speedup vs baseline: 1.1399x; 1.0254x over previous
import functools

import jax
import jax.numpy as jnp
from jax import lax
from jax.experimental import pallas as pl
from jax.experimental.pallas import tpu as pltpu
from jax.experimental.pallas import tpu_sc as plsc

_N = 10000
_E = 320000
_D = 128

_NSC = 2
_NSUB = 16
_NW = _NSC * _NSUB
_CH = 80
_NCHK = _E // _CH
_CPT = _NCHK // _NW
_LEFT = _NCHK - _CPT * _NW
_DCH = 128
_DCPT = (_E // _DCH) // _NW
_DLEFT = _E // _DCH - _DCPT * _NW
_RPT = 624

_mesh = plsc.VectorSubcoreMesh(core_axis_name="c", subcore_axis_name="s")


def _zero_my_slice(zer_v, acc_sh, sid, sem):
    r0 = sid * _RPT
    cs = [pltpu.make_async_copy(zer_v, acc_sh.at[pl.ds(r0 + j * 48, 48)], sem)
          for j in range(13)]
    for c in cs:
        c.start()

    @pl.when(sid == _NSUB - 1)
    def _():
        pltpu.sync_copy(zer_v.at[pl.ds(0, 16)], acc_sh.at[pl.ds(_N - 16, 16)])

    for c in cs:
        c.wait()


def _write_my_slice(acc_sh, out_hbm, cid, sid, sem):
    r0 = sid * _RPT
    cs = [pltpu.make_async_copy(acc_sh.at[pl.ds(r0 + j * 48, 48)],
                                out_hbm.at[cid, pl.ds(r0 + j * 48, 48), :],
                                sem)
          for j in range(13)]
    for c in cs:
        c.start()

    @pl.when(sid == _NSUB - 1)
    def _():
        pltpu.sync_copy(acc_sh.at[pl.ds(_N - 16, 16)],
                        out_hbm.at[cid, pl.ds(_N - 16, 16), :])

    for c in cs:
        c.wait()


@functools.partial(
    pl.kernel,
    out_type=jax.ShapeDtypeStruct((_NSC, _N, 16), jnp.float32),
    mesh=_mesh,
    scratch_types=[
        pltpu.VMEM((_DCH,), jnp.int32),
        pltpu.VMEM((_DCH,), jnp.int32),
        pltpu.VMEM((_DCH,), jnp.int32),
        pltpu.VMEM((_DCH,), jnp.int32),
        pltpu.VMEM((_DCH, 16), jnp.float32),
        pltpu.VMEM((48, 16), jnp.float32),
        pltpu.VMEM_SHARED((_N, 16), jnp.float32),
        pltpu.SemaphoreType.DMA,
        pltpu.SemaphoreType.DMA,
        pltpu.SemaphoreType.DMA,
        pltpu.SemaphoreType.DMA,
        pltpu.SemaphoreType.DMA,
        pltpu.SemaphoreType.DMA,
        pltpu.SemaphoreType.DMA,
        pltpu.SemaphoreType.DMA,
    ],
)
def _deg_kernel(dst_hbm, out_hbm, di0, di1, di2, di3, ones_v, zer_v, acc_sh,
                semI0, semI1, semI2, semI3, semS0, semS1, semS2, semS3):
    cid = lax.axis_index("c")
    sid = lax.axis_index("s")
    wid = sid * _NSC + cid
    base = wid * _DCPT

    dis = (di0, di1, di2, di3)
    semIs = (semI0, semI1, semI2, semI3)
    semSs = (semS0, semS1, semS2, semS3)

    def idx(c, k):
        return pltpu.make_async_copy(
            dst_hbm.at[pl.ds(c * _DCH, _DCH)], dis[k], semIs[k])

    def scat(k):
        return pltpu.make_async_copy(ones_v, acc_sh.at[dis[k]], semSs[k])

    idx(base, 0).start()
    idx(base + 1, 1).start()

    @pl.loop(0, 48)
    def _(i):
        zer_v[i, :] = jnp.zeros((16,), jnp.float32)

    @pl.loop(0, _DCH)
    def _(i):
        ones_v[i, :] = jnp.ones((16,), jnp.float32)

    _zero_my_slice(zer_v, acc_sh, sid, semS3)
    plsc.subcore_barrier()

    def part(c, k, wait_prev2):
        idx(c, k).wait()
        scat(k).start(add=True)
        if wait_prev2:
            scat((k - 2) % 4).wait()

        @pl.when(c + 2 < base + _DCPT)
        def _():
            idx(c + 2, (k + 2) % 4).start()

    part(base, 0, False)
    part(base + 1, 1, False)

    @pl.loop(0, (_DCPT - 2) // 4)
    def _(u):
        c0 = base + 2 + u * 4
        part(c0, 2, True)
        part(c0 + 1, 3, True)
        part(c0 + 2, 0, True)
        part(c0 + 3, 1, True)

    scat(0).wait()
    scat(1).wait()

    @pl.when(wid < _DLEFT)
    def _():
        c = _NW * _DCPT + wid
        idx(c, 0).start()
        idx(c, 0).wait()
        scat(0).start(add=True)
        scat(0).wait()

    plsc.subcore_barrier()
    _write_my_slice(acc_sh, out_hbm, cid, sid, semS3)


@functools.partial(
    pl.kernel,
    out_type=jax.ShapeDtypeStruct((_NSC, _N, _D), jnp.float32),
    mesh=_mesh,
    scratch_types=[
        pltpu.VMEM((_CH,), jnp.int32),
        pltpu.VMEM((_CH,), jnp.int32),
        pltpu.VMEM((_CH,), jnp.int32),
        pltpu.VMEM((_CH,), jnp.int32),
        pltpu.VMEM((_CH,), jnp.int32),
        pltpu.VMEM((_CH,), jnp.int32),
        pltpu.VMEM((_CH,), jnp.int32),
        pltpu.VMEM((_CH,), jnp.int32),
        pltpu.VMEM((_CH, _D), jnp.float32),
        pltpu.VMEM((_CH, _D), jnp.float32),
        pltpu.VMEM((_CH, _D), jnp.float32),
        pltpu.VMEM((_CH, _D), jnp.float32),
        pltpu.VMEM_SHARED((_N, _D), jnp.float32),
        pltpu.SemaphoreType.DMA,
        pltpu.SemaphoreType.DMA,
        pltpu.SemaphoreType.DMA,
        pltpu.SemaphoreType.DMA,
        pltpu.SemaphoreType.DMA,
        pltpu.SemaphoreType.DMA,
        pltpu.SemaphoreType.DMA,
        pltpu.SemaphoreType.DMA,
        pltpu.SemaphoreType.DMA,
        pltpu.SemaphoreType.DMA,
        pltpu.SemaphoreType.DMA,
        pltpu.SemaphoreType.DMA,
        pltpu.SemaphoreType.DMA,
        pltpu.SemaphoreType.DMA,
        pltpu.SemaphoreType.DMA,
        pltpu.SemaphoreType.DMA,
    ],
)
def _segsum_kernel(hs_hbm, src_hbm, dst_hbm, out_hbm,
                   si0, si1, si2, si3, di0, di1, di2, di3,
                   r0_v, r1_v, r2_v, r3_v, acc_sh,
                   semIS0, semIS1, semIS2, semIS3,
                   semID0, semID1, semID2, semID3,
                   semG0, semG1, semG2, semG3,
                   semS0, semS1, semS2, semS3):
    cid = lax.axis_index("c")
    sid = lax.axis_index("s")
    wid = sid * _NSC + cid
    base = wid * _CPT
    end = base + _CPT

    sis = (si0, si1, si2, si3)
    dis = (di0, di1, di2, di3)
    rows = (r0_v, r1_v, r2_v, r3_v)
    semISs = (semIS0, semIS1, semIS2, semIS3)
    semIDs = (semID0, semID1, semID2, semID3)
    semGs = (semG0, semG1, semG2, semG3)
    semSs = (semS0, semS1, semS2, semS3)

    def isrc_k(c, k):
        return pltpu.make_async_copy(
            src_hbm.at[pl.ds(c * _CH, _CH)], sis[k], semISs[k])

    def idst_k(c, k):
        return pltpu.make_async_copy(
            dst_hbm.at[pl.ds(c * _CH, _CH)], dis[k], semIDs[k])

    def gat(k):
        return pltpu.make_async_copy(hs_hbm.at[sis[k]], rows[k], semGs[k])

    def scat(k):
        return pltpu.make_async_copy(rows[k], acc_sh.at[dis[k]], semSs[k])

    isrc_k(base, 0).start()
    idst_k(base, 0).start()
    isrc_k(base + 1, 1).start()
    idst_k(base + 1, 1).start()

    @pl.loop(0, _CH)
    def _(i):
        @pl.loop(0, _D // 16)
        def _(j):
            r3_v[i, pl.ds(j * 16, 16)] = jnp.zeros((16,), jnp.float32)

    rbase = sid * _RPT
    zcs = [pltpu.make_async_copy(
               r3_v, acc_sh.at[pl.ds(rbase + j * _CH, _CH)], semS3)
           for j in range(7)]
    zcs.append(pltpu.make_async_copy(
        r3_v.at[pl.ds(0, 64)], acc_sh.at[pl.ds(rbase + 7 * _CH, 64)], semS3))
    for z in zcs:
        z.start()

    @pl.when(sid == _NSUB - 1)
    def _():
        pltpu.sync_copy(r3_v.at[pl.ds(0, 16)], acc_sh.at[pl.ds(_N - 16, 16)])

    for z in zcs:
        z.wait()

    isrc_k(base, 0).wait()
    gat(0).start()
    isrc_k(base + 1, 1).wait()
    gat(1).start()
    isrc_k(base + 2, 2).start()
    isrc_k(base + 3, 3).start()
    plsc.subcore_barrier()

    def part(c, k, first, s4, w2):
        gat(k).wait()
        if not first:
            scat((k + 2) % 4).wait()
        if w2:
            @pl.when(c + 2 < end)
            def _():
                idst_k(c + 2, (k + 2) % 4).start()
                isrc_k(c + 2, (k + 2) % 4).wait()
                gat((k + 2) % 4).start()
        if s4:
            @pl.when(c + 4 < end)
            def _():
                isrc_k(c + 4, k).start()
        idst_k(c, k).wait()
        scat(k).start(add=True)

    part(base + 0, 0, True, True, True)
    part(base + 1, 1, True, True, True)

    @pl.loop(0, (_CPT - 5) // 4)
    def _(u):
        c0 = base + 2 + u * 4
        part(c0, 2, False, True, True)
        part(c0 + 1, 3, False, True, True)
        part(c0 + 2, 0, False, True, True)
        part(c0 + 3, 1, False, True, True)

    part(end - 3, 2, False, False, True)
    part(end - 2, 3, False, False, False)
    part(end - 1, 0, False, False, False)

    scat(3).wait()
    scat(0).wait()

    plsc.subcore_barrier()
    _write_my_slice(acc_sh, out_hbm, cid, sid, semS3)


_BLK = 1000


def _tc1_body(x_ref, lx_ref, w_ref, b_ref, xo_ref, h_ref):
    xo = x_ref[...] * lx_ref[...]
    xo_ref[...] = xo
    h_ref[...] = (jnp.dot(xo, w_ref[...], preferred_element_type=jnp.float32)
                  + b_ref[...])


def _tc2_body(d0_ref, d1_ref, h_ref, hs_ref):
    dinv = lax.rsqrt(d0_ref[...] + d1_ref[...] + 1.0)
    hs_ref[...] = h_ref[...] * dinv


def _tc3_body(d0_ref, d1_ref, s0_ref, s1_ref, hs_ref, w_ref, b_ref, out_ref):
    dinv = lax.rsqrt(d0_ref[...] + d1_ref[...] + 1.0)
    t = (s0_ref[...] + s1_ref[...] + hs_ref[...]) * dinv
    h1 = jnp.maximum(t, 0.0)
    out_ref[...] = (jnp.dot(h1, w_ref[...], preferred_element_type=jnp.float32)
                    + b_ref[...]) * dinv


def _tc4_body(d0_ref, d1_ref, s0_ref, s1_ref, hs_ref,
              w1_ref, b1_ref, w2_ref, b2_ref, out_ref):
    dinv = lax.rsqrt(d0_ref[...] + d1_ref[...] + 1.0)
    embed = (s0_ref[...] + s1_ref[...] + hs_ref[...]) * dinv
    hidden = jnp.maximum(
        jnp.dot(embed, w1_ref[...], preferred_element_type=jnp.float32)
        + b1_ref[...], 0.0)
    out_ref[...] = (jnp.dot(hidden, w2_ref[...],
                            preferred_element_type=jnp.float32) + b2_ref[...])


def _row_spec():
    return pl.BlockSpec((_BLK, _D), lambda i: (i, 0))


def _deg_spec():
    return pl.BlockSpec((_BLK, 1), lambda i: (i, 0))


def _full_spec(shape):
    return pl.BlockSpec(shape, lambda i: tuple(0 for _ in shape))


def kernel(x, edge_index, emb1, emb3, learnable_x, cond_Wi, cond_bi, cond_Wo,
           cond_bo, g_W1, g_b1, g_W2, g_b2, c_W1, c_b1, c_W2, c_b2):
    src = edge_index[0]
    dst = edge_index[1]
    n, d = x.shape
    grid = (n // _BLK,)

    degp = _deg_kernel(dst)
    d0 = degp[0, :, 0].reshape(n, 1)
    d1 = degp[1, :, 0].reshape(n, 1)

    x_out, h1_pre = pl.pallas_call(
        _tc1_body,
        grid=grid,
        in_specs=[_row_spec(), _row_spec(),
                  _full_spec((_D, _D)), _full_spec((1, _D))],
        out_specs=[_row_spec(), _row_spec()],
        out_shape=[jax.ShapeDtypeStruct((n, d), jnp.float32)] * 2,
    )(x, learnable_x, g_W1, g_b1.reshape(1, d))

    hs1 = pl.pallas_call(
        _tc2_body,
        grid=grid,
        in_specs=[_deg_spec(), _deg_spec(), _row_spec()],
        out_specs=_row_spec(),
        out_shape=jax.ShapeDtypeStruct((n, d), jnp.float32),
    )(d0, d1, h1_pre)

    s1 = _segsum_kernel(hs1, src, dst)

    hs2 = pl.pallas_call(
        _tc3_body,
        grid=grid,
        in_specs=[_deg_spec(), _deg_spec(), _row_spec(), _row_spec(),
                  _row_spec(), _full_spec((_D, _D)), _full_spec((1, _D))],
        out_specs=_row_spec(),
        out_shape=jax.ShapeDtypeStruct((n, d), jnp.float32),
    )(d0, d1, s1[0], s1[1], hs1, g_W2, g_b2.reshape(1, d))

    s2 = _segsum_kernel(hs2, src, dst)

    nh = c_W1.shape[1]
    nc = c_W2.shape[1]
    logits = pl.pallas_call(
        _tc4_body,
        grid=grid,
        in_specs=[_deg_spec(), _deg_spec(), _row_spec(), _row_spec(),
                  _row_spec(), _full_spec((_D, nh)), _full_spec((1, nh)),
                  _full_spec((nh, nc)), _full_spec((1, nc))],
        out_specs=pl.BlockSpec((_BLK, nc), lambda i: (i, 0)),
        out_shape=jax.ShapeDtypeStruct((n, nc), jnp.float32),
    )(d0, d1, s2[0], s2[1], hs2, c_W1, c_b1.reshape(1, nh),
      c_W2, c_b2.reshape(1, nc))

    return (x_out, logits)
